# initial kernel scaffold (unmeasured)
import jax
import jax.numpy as jnp
from jax import lax
from jax.experimental import pallas as pl
from jax.experimental.pallas import tpu as pltpu


def kernel(
    x,
):
    def body(*refs):
        pass

    out_shape = jax.ShapeDtypeStruct(..., jnp.float32)
    return pl.pallas_call(body, out_shape=out_shape)(...)


# baseline (device time: 49667 ns/iter reference)
import jax
import jax.numpy as jnp
from jax import lax
from jax.experimental import pallas as pl
from jax.experimental.pallas import tpu as pltpu

N_DEV = 4
CHUNK = 256


def kernel(x):
    m, n = x.shape
    n_chunks = m // CHUNK

    def body(x_ref, out_ref, carry_ref, send_buf, recv_buf, send_sem, recv_sem):
        my = lax.axis_index("i")

        total = jnp.sum(x_ref[...], axis=0, keepdims=True)

        @pl.when(my == 0)
        def _():
            carry_ref[...] = jnp.zeros((1, n), jnp.float32)

        @pl.when(my > 0)
        def _():
            recv = pltpu.make_async_remote_copy(
                src_ref=send_buf,
                dst_ref=recv_buf,
                send_sem=send_sem,
                recv_sem=recv_sem,
                device_id=(my - 1,),
                device_id_type=pl.DeviceIdType.MESH,
            )
            recv.wait_recv()
            carry_ref[...] = recv_buf[...]

        @pl.when(my < N_DEV - 1)
        def _():
            send_buf[...] = carry_ref[...] + total
            send = pltpu.make_async_remote_copy(
                src_ref=send_buf,
                dst_ref=recv_buf,
                send_sem=send_sem,
                recv_sem=recv_sem,
                device_id=(my + 1,),
                device_id_type=pl.DeviceIdType.MESH,
            )
            send.start()
            send.wait_send()

        row = lax.broadcasted_iota(jnp.int32, (CHUNK, CHUNK), 0)
        col = lax.broadcasted_iota(jnp.int32, (CHUNK, CHUNK), 1)
        tri = (row >= col).astype(jnp.bfloat16)

        def step(i, _):
            chunk = x_ref[pl.ds(i * CHUNK, CHUNK), :].astype(jnp.bfloat16)
            cum = lax.dot_general(
                tri, chunk,
                (((1,), (0,)), ((), ())),
                preferred_element_type=jnp.float32,
            )
            cum = cum + carry_ref[...]
            out_ref[pl.ds(i * CHUNK, CHUNK), :] = cum.astype(out_ref.dtype)
            carry_ref[...] = cum[CHUNK - 1:CHUNK, :]
            return 0

        lax.fori_loop(0, n_chunks, step, 0)

    return pl.pallas_call(
        body,
        out_shape=jax.ShapeDtypeStruct((m, n), jnp.bfloat16),
        in_specs=[pl.BlockSpec(memory_space=pltpu.VMEM)],
        out_specs=pl.BlockSpec(memory_space=pltpu.VMEM),
        scratch_shapes=[
            pltpu.VMEM((1, n), jnp.float32),
            pltpu.VMEM((1, n), jnp.float32),
            pltpu.VMEM((1, n), jnp.float32),
            pltpu.SemaphoreType.DMA,
            pltpu.SemaphoreType.DMA,
        ],
        compiler_params=pltpu.CompilerParams(vmem_limit_bytes=60 * 1024 * 1024),
    )(x)
